# baseline (speedup 1.0000x reference)
import jax
import jax.numpy as jnp
from jax.experimental import pallas as pl
from jax.experimental.pallas import tpu as pltpu

_CHUNK_ROWS = 10000
_SLOTS = 9


def kernel(embed_user, embed_item):
    n, d = embed_user.shape
    chunk = _CHUNK_ROWS if n % _CHUNK_ROWS == 0 else n
    nchunks = n // chunk
    total = 2 * nchunks
    slots = min(_SLOTS, total)

    def body(user_hbm, item_hbm, out_hbm, buf, *sems):
        sem_in, sem_out = sems[:slots], sems[slots:]
        srcs = (user_hbm, item_hbm)

        for lo in range(0, total, slots):
            ks = range(lo, min(lo + slots, total))
            loads, stores = [], []
            for k in ks:
                t, c = divmod(k, nchunks)
                p = k - lo
                loads.append(pltpu.make_async_copy(
                    srcs[t].at[pl.ds(c * chunk, chunk)], buf.at[p], sem_in[p]))
                stores.append(pltpu.make_async_copy(
                    buf.at[p], out_hbm.at[t, pl.ds(c * chunk, chunk)],
                    sem_out[p]))
            for ld in loads:
                ld.start()
            for ld in loads:
                ld.wait()
            for st in stores:
                st.start()
            for st in stores:
                st.wait()

    return pl.pallas_call(
        body,
        out_shape=jax.ShapeDtypeStruct((2, n, d), embed_user.dtype),
        in_specs=[
            pl.BlockSpec(memory_space=pltpu.MemorySpace.HBM),
            pl.BlockSpec(memory_space=pltpu.MemorySpace.HBM),
        ],
        out_specs=pl.BlockSpec(memory_space=pltpu.MemorySpace.HBM),
        scratch_shapes=(
            [pltpu.VMEM((slots, chunk, d), embed_user.dtype)]
            + [pltpu.SemaphoreType.DMA] * (2 * slots)
        ),
    )(embed_user, embed_item)


# speedup vs baseline: 1.0256x; 1.0256x over previous
import jax
import jax.numpy as jnp
from jax.experimental import pallas as pl
from jax.experimental.pallas import tpu as pltpu

_CHUNK_ROWS = 10000
_SLOTS = 10


def kernel(embed_user, embed_item):
    n, d = embed_user.shape
    chunk = _CHUNK_ROWS if n % _CHUNK_ROWS == 0 else n
    nchunks = n // chunk
    total = 2 * nchunks
    slots = min(_SLOTS, total)

    def body(user_hbm, item_hbm, out_hbm, buf, *sems):
        sem_in, sem_out = sems[:slots], sems[slots:]
        srcs = (user_hbm, item_hbm)

        for lo in range(0, total, slots):
            ks = range(lo, min(lo + slots, total))
            loads, stores = [], []
            for k in ks:
                t, c = divmod(k, nchunks)
                p = k - lo
                loads.append(pltpu.make_async_copy(
                    srcs[t].at[pl.ds(c * chunk, chunk)], buf.at[p], sem_in[p]))
                stores.append(pltpu.make_async_copy(
                    buf.at[p], out_hbm.at[t, pl.ds(c * chunk, chunk)],
                    sem_out[p]))
            for ld in loads:
                ld.start()
            for ld in loads:
                ld.wait()
            for st in stores:
                st.start()
            for st in stores:
                st.wait()

    return pl.pallas_call(
        body,
        out_shape=jax.ShapeDtypeStruct((2, n, d), embed_user.dtype),
        in_specs=[
            pl.BlockSpec(memory_space=pltpu.MemorySpace.HBM),
            pl.BlockSpec(memory_space=pltpu.MemorySpace.HBM),
        ],
        out_specs=pl.BlockSpec(memory_space=pltpu.MemorySpace.HBM),
        scratch_shapes=(
            [pltpu.VMEM((slots, chunk, d), embed_user.dtype)]
            + [pltpu.SemaphoreType.DMA] * (2 * slots)
        ),
    )(embed_user, embed_item)


# speedup vs baseline: 1.0257x; 1.0001x over previous
import jax
import jax.numpy as jnp
from jax.experimental import pallas as pl
from jax.experimental.pallas import tpu as pltpu

_CHUNK_ROWS = 10000
_SLOTS = 10


def kernel(embed_user, embed_item):
    n, d = embed_user.shape
    chunk = _CHUNK_ROWS if n % _CHUNK_ROWS == 0 else n
    nchunks = n // chunk
    total = 2 * nchunks
    slots = min(_SLOTS, total)

    def body(user_hbm, item_hbm, out_hbm, buf, *sems):
        sem_in, sem_out = sems[:slots], sems[slots:]
        srcs = (user_hbm, item_hbm)

        for lo in range(0, total, slots):
            ks = range(lo, min(lo + slots, total))
            loads, stores = [], []
            for k in ks:
                t, c = k % 2, k // 2
                p = k - lo
                loads.append(pltpu.make_async_copy(
                    srcs[t].at[pl.ds(c * chunk, chunk)], buf.at[p], sem_in[p]))
                stores.append(pltpu.make_async_copy(
                    buf.at[p], out_hbm.at[t, pl.ds(c * chunk, chunk)],
                    sem_out[p]))
            for ld in loads:
                ld.start()
            for ld in loads:
                ld.wait()
            for st in stores:
                st.start()
            for st in stores:
                st.wait()

    return pl.pallas_call(
        body,
        out_shape=jax.ShapeDtypeStruct((2, n, d), embed_user.dtype),
        in_specs=[
            pl.BlockSpec(memory_space=pltpu.MemorySpace.HBM),
            pl.BlockSpec(memory_space=pltpu.MemorySpace.HBM),
        ],
        out_specs=pl.BlockSpec(memory_space=pltpu.MemorySpace.HBM),
        scratch_shapes=(
            [pltpu.VMEM((slots, chunk, d), embed_user.dtype)]
            + [pltpu.SemaphoreType.DMA] * (2 * slots)
        ),
    )(embed_user, embed_item)


# speedup vs baseline: 1.0442x; 1.0180x over previous
import jax
import jax.numpy as jnp
from jax.experimental import pallas as pl

_MAX_BLOCK_ROWS = 10000


def _pick_block_rows(n):
    best = 0
    for bn in range(8, min(_MAX_BLOCK_ROWS, n) + 1, 8):
        if n % bn == 0:
            best = bn
    return best if best else n


def _copy_body(user_ref, item_ref, out_ref):
    out_ref[0] = user_ref[...]
    out_ref[1] = item_ref[...]


def kernel(embed_user, embed_item):
    n, d = embed_user.shape
    bn = _pick_block_rows(n)
    return pl.pallas_call(
        _copy_body,
        grid=(n // bn,),
        in_specs=[
            pl.BlockSpec((bn, d), lambda j: (j, 0)),
            pl.BlockSpec((bn, d), lambda j: (j, 0)),
        ],
        out_specs=pl.BlockSpec((2, bn, d), lambda j: (0, j, 0)),
        out_shape=jax.ShapeDtypeStruct((2, n, d), embed_user.dtype),
    )(embed_user, embed_item)
